# Initial kernel scaffold; baseline (speedup 1.0000x reference)
import jax
import jax.numpy as jnp
from jax.experimental import pallas as pl


def kernel(x, edge_attr, e1W, e1b, e2W, e2b, e3W, e3b, e4W, e4b, root1, bias1, root2, bias2, root3, bias3, root4, bias4, fc1W, fc1b, fc2W, fc2b, edge_index, batch):
    raise NotImplementedError("write your pallas kernel here")


# speedup vs baseline: 2.8558x; 2.8558x over previous
import functools
import jax
import jax.numpy as jnp
from jax import lax
from jax.experimental import pallas as pl
from jax.experimental.pallas import tpu as pltpu
from jax.experimental.pallas import tpu_sc as plsc

_N = 10000
_E = 20000
_G = 500
_D = 128
_NW = 32
_EPT = 640
_EP = _NW * _EPT
_NCH = 5
_CH = 128
_ACC_ROWS = 10240
_ZPT = 640
_NP = 10240
_PNG = 16
_PSUB = 320

_mesh = plsc.VectorSubcoreMesh(core_axis_name="c", subcore_axis_name="s")


def _sc_gather(table, idx3):
  @functools.partial(
      pl.kernel, mesh=_mesh,
      out_type=jax.ShapeDtypeStruct((_EP, _D), jnp.float32),
      scratch_types=[
          pltpu.VMEM((_NCH, _CH), jnp.int32),
          pltpu.VMEM((_EPT, _D), jnp.float32),
          pltpu.SemaphoreType.DMA,
      ])
  def k(table_hbm, idx_hbm, out_hbm, idx_v, rows_v, sem):
    wid = lax.axis_index("s") * 2 + lax.axis_index("c")
    pltpu.sync_copy(idx_hbm.at[wid], idx_v)
    copies = []
    for j in range(_NCH):
      copies.append(pltpu.async_copy(
          table_hbm.at[idx_v.at[j]], rows_v.at[pl.ds(j * _CH, _CH)], sem))
    for cp in copies:
      cp.wait()
    pltpu.sync_copy(rows_v, out_hbm.at[pl.ds(wid * _EPT, _EPT)])

  return k(table, idx3)


_HROWS = 5120
_H0 = _HROWS + 128


def _sc_scatter_add(vals, idxs, zrows, nrows):
  @functools.partial(
      pl.kernel, mesh=_mesh,
      out_type=jax.ShapeDtypeStruct((nrows, _D), jnp.float32),
      scratch_types=[
          pltpu.VMEM((2 * _NCH, _CH), jnp.int32),
          pltpu.VMEM((_EPT, _D), jnp.float32),
          pltpu.VMEM_SHARED((nrows, _D), jnp.float32),
      ])
  def k(vals_hbm, idx_hbm, z_hbm, out_hbm, idx_v, vals_v, acc):
    c = lax.axis_index("c")
    s = lax.axis_index("s")
    zpt = nrows // 16

    @pl.when(c == 0)
    def _():
      pltpu.sync_copy(idx_hbm.at[s], idx_v)
      pltpu.sync_copy(z_hbm.at[pl.ds(0, zpt)], acc.at[pl.ds(s * zpt, zpt)])
      plsc.subcore_barrier()
      for r in range(2):
        pltpu.sync_copy(
            vals_hbm.at[pl.ds(s * 2 * _EPT + r * _EPT, _EPT)], vals_v)
        for j in range(_NCH):
          pltpu.sync_copy(vals_v.at[pl.ds(j * _CH, _CH)],
                          acc.at[idx_v.at[r * _NCH + j]], add=True)
      plsc.subcore_barrier()
      pltpu.sync_copy(acc.at[pl.ds(s * zpt, zpt)],
                      out_hbm.at[pl.ds(s * zpt, zpt)])

  return k(vals, idxs, zrows)


def _edge_tc(xs, ea8, wstack, extra8):
  be = 2048

  def body(xs_ref, ea_ref, w_ref, ex_ref, out_ref):
    xsb = xs_ref[...]
    acc = jnp.dot(xsb, w_ref[4], preferred_element_type=jnp.float32)
    for kk in range(4):
      zk = jnp.dot(xsb, w_ref[kk], preferred_element_type=jnp.float32)
      acc = acc + ea_ref[:, kk:kk + 1] * zk
    acc = acc + ex_ref[0:1, :]
    out_ref[...] = acc

  return pl.pallas_call(
      body,
      grid=(_EP // be,),
      in_specs=[
          pl.BlockSpec((be, _D), lambda i: (i, 0)),
          pl.BlockSpec((be, 8), lambda i: (i, 0)),
          pl.BlockSpec((5, _D, _D), lambda i: (0, 0, 0)),
          pl.BlockSpec((8, _D), lambda i: (0, 0)),
      ],
      out_specs=pl.BlockSpec((be, _D), lambda i: (i, 0)),
      out_shape=jax.ShapeDtypeStruct((_EP, _D), jnp.float32),
  )(xs, ea8, wstack, extra8)


def _node_tc(s3, rin16, x, root, bias8, mask8, first):
  bn = 2000

  def body(sr, rr16, xr, rootr, br, mr, outr, rout):
    s = sr[...]
    if first:
      cnt = s[:, 112:113]
      rin = 1.0 / jnp.maximum(cnt, 1.0)
    else:
      rin = rr16[:, 0:1]
    z = jnp.dot(xr[...], rootr[...], preferred_element_type=jnp.float32)
    h = jnp.maximum(s * rin + z + br[0:1, :], 0.0)
    outr[...] = h * mr[0:1, :]
    rout[...] = jnp.broadcast_to(rin, (bn, 16))

  return pl.pallas_call(
      body,
      grid=(_N // bn,),
      in_specs=[
          pl.BlockSpec((bn, _D), lambda i: (i, 0)),
          pl.BlockSpec((bn, 16), lambda i: (i, 0)),
          pl.BlockSpec((bn, _D), lambda i: (i, 0)),
          pl.BlockSpec((_D, _D), lambda i: (0, 0)),
          pl.BlockSpec((8, _D), lambda i: (0, 0)),
          pl.BlockSpec((8, _D), lambda i: (0, 0)),
      ],
      out_specs=[
          pl.BlockSpec((bn, _D), lambda i: (i, 0)),
          pl.BlockSpec((bn, 16), lambda i: (i, 0)),
      ],
      out_shape=[
          jax.ShapeDtypeStruct((_N, _D), jnp.float32),
          jax.ShapeDtypeStruct((_N, 16), jnp.float32),
      ],
  )(s3, rin16, x, root, bias8, mask8)


def _pool_fc_tc(h, batch, w1, b18, w2, b28):
  def body(b_ref, h_ref, w1_ref, b1_ref, w2_ref, b2_ref, out_ref, acc):
    acc[...] = jnp.zeros_like(acc)

    def step(i, c):
      g = b_ref[i]
      row = h_ref[pl.ds(i, 1), :]
      acc[pl.ds(g, 1), :] = jnp.maximum(acc[pl.ds(g, 1), :], row)
      return c

    lax.fori_loop(0, _N, step, 0)
    p = acc[0:_G, :]
    hh = jnp.maximum(
        jnp.dot(p, w1_ref[...], preferred_element_type=jnp.float32)
        + b1_ref[0:1, :], 0.0)
    out_ref[...] = jnp.dot(hh, w2_ref[...],
                           preferred_element_type=jnp.float32) + b2_ref[0:1, :]

  return pl.pallas_call(
      body,
      grid=(1,),
      in_specs=[
          pl.BlockSpec(memory_space=pltpu.SMEM),
          pl.BlockSpec((_N, _D), lambda i: (0, 0)),
          pl.BlockSpec((_D, 64), lambda i: (0, 0)),
          pl.BlockSpec((8, 64), lambda i: (0, 0)),
          pl.BlockSpec((64, 10), lambda i: (0, 0)),
          pl.BlockSpec((8, 10), lambda i: (0, 0)),
      ],
      out_specs=pl.BlockSpec((_G, 10), lambda i: (0, 0)),
      out_shape=jax.ShapeDtypeStruct((_G, 10), jnp.float32),
      scratch_shapes=[pltpu.VMEM((512, _D), jnp.float32)],
  )(batch, h, w1, b18, w2, b28)


def kernel(x, edge_attr, e1W, e1b, e2W, e2b, e3W, e3b, e4W, e4b,
           root1, bias1, root2, bias2, root3, bias3, root4, bias4,
           fc1W, fc1b, fc2W, fc2b, edge_index, batch):
  dims = [(128, 32), (32, 48), (48, 64), (64, 128)]
  eWs = [e1W, e2W, e3W, e4W]
  ebs = [e1b, e2b, e3b, e4b]
  roots = [root1, root2, root3, root4]
  biases = [bias1, bias2, bias3, bias4]

  src = edge_index[0]
  dst = edge_index[1]
  pad = _EP - _E
  src_p = jnp.concatenate([src, jnp.zeros((pad,), jnp.int32)])
  dst_p = jnp.concatenate([dst, jnp.full((pad,), _N, jnp.int32)])
  idx_src = src_p.reshape(_NW, _NCH, _CH)
  idx_h0 = jnp.where(dst_p < _HROWS, dst_p,
                     _HROWS).reshape(16, 2 * _NCH, _CH)
  idx_h1 = jnp.where(dst_p >= _HROWS, dst_p - _HROWS,
                     _N - _HROWS).reshape(16, 2 * _NCH, _CH)
  ea8 = jnp.concatenate(
      [edge_attr, jnp.zeros((pad, 4), jnp.float32)], axis=0)
  ea8 = jnp.concatenate([ea8, jnp.zeros((_EP, 4), jnp.float32)], axis=1)
  zrows = jnp.zeros((_EPT, _D), jnp.float32)

  h = x
  rin16 = jnp.zeros((_N, 16), jnp.float32)
  for i, (ci, co) in enumerate(dims):
    wstack = jnp.zeros((5, _D, _D), jnp.float32).at[:, :ci, :co].set(
        jnp.concatenate(
            [eWs[i].reshape(4, ci, co), ebs[i].reshape(1, ci, co)], axis=0))
    rootp = jnp.zeros((_D, _D), jnp.float32).at[:ci, :co].set(roots[i])
    bias8 = jnp.broadcast_to(
        jnp.pad(biases[i], (0, _D - co)), (8, _D))
    mask8 = jnp.broadcast_to(
        (jnp.arange(_D) < co).astype(jnp.float32), (8, _D))
    extra8 = jnp.broadcast_to(
        ((jnp.arange(_D) >= 112) & (i == 0)).astype(jnp.float32), (8, _D))

    xs = _sc_gather(h, idx_src)
    msg = _edge_tc(xs, ea8, wstack, extra8)
    s0 = _sc_scatter_add(msg, idx_h0, zrows, _H0)
    s1 = _sc_scatter_add(msg, idx_h1, zrows, _HROWS)
    s3 = jnp.concatenate([s0[:_HROWS], s1], axis=0)
    h, rin16 = _node_tc(s3, rin16, h, rootp, bias8, mask8, first=(i == 0))

  return _pool_fc_tc(h, batch, fc1W, jnp.broadcast_to(fc1b, (8, 64)),
                     fc2W, jnp.broadcast_to(fc2b, (8, 10)))


# speedup vs baseline: 3.4814x; 1.2190x over previous
import functools
import jax
import jax.numpy as jnp
from jax import lax
from jax.experimental import pallas as pl
from jax.experimental.pallas import tpu as pltpu
from jax.experimental.pallas import tpu_sc as plsc

_N = 10000
_E = 20000
_G = 500
_D = 128
_NW = 32
_EPT = 640
_EP = _NW * _EPT
_NCH = 5
_CH = 128
_ACC_ROWS = 10240
_ZPT = 640
_NP = 10240
_PNG = 16
_PSUB = 320

_mesh = plsc.VectorSubcoreMesh(core_axis_name="c", subcore_axis_name="s")


def _sc_gather(table, idx3):
  @functools.partial(
      pl.kernel, mesh=_mesh,
      out_type=jax.ShapeDtypeStruct((_EP, _D), jnp.float32),
      scratch_types=[
          pltpu.VMEM((_NCH, _CH), jnp.int32),
          pltpu.VMEM((_EPT, _D), jnp.float32),
          pltpu.SemaphoreType.DMA,
      ])
  def k(table_hbm, idx_hbm, out_hbm, idx_v, rows_v, sem):
    wid = lax.axis_index("s") * 2 + lax.axis_index("c")
    pltpu.sync_copy(idx_hbm.at[wid], idx_v)
    copies = []
    for j in range(_NCH):
      copies.append(pltpu.async_copy(
          table_hbm.at[idx_v.at[j]], rows_v.at[pl.ds(j * _CH, _CH)], sem))
    for cp in copies:
      cp.wait()
    pltpu.sync_copy(rows_v, out_hbm.at[pl.ds(wid * _EPT, _EPT)])

  return k(table, idx3)


_HROWS = 5120
_H0 = _HROWS + 128


_HROWS = 5120
_HACC = 5248


def _sc_scatter_add(vals, idxs, zrows):
  @functools.partial(
      pl.kernel, mesh=_mesh,
      out_type=jax.ShapeDtypeStruct((2, _HACC, _D), jnp.float32),
      scratch_types=[
          pltpu.VMEM((2 * _NCH, _CH), jnp.int32),
          pltpu.VMEM((_EPT, _D), jnp.float32),
          pltpu.VMEM_SHARED((_HACC, _D), jnp.float32),
      ])
  def k(vals_hbm, idx_hbm, z_hbm, out_hbm, idx_v, vals_v, acc):
    c = lax.axis_index("c")
    s = lax.axis_index("s")
    zpt = _HACC // 16
    pltpu.sync_copy(idx_hbm.at[c, s], idx_v)
    pltpu.sync_copy(z_hbm.at[pl.ds(0, zpt)], acc.at[pl.ds(s * zpt, zpt)])
    plsc.subcore_barrier()
    for r in range(2):
      pltpu.sync_copy(
          vals_hbm.at[pl.ds(s * 2 * _EPT + r * _EPT, _EPT)], vals_v)
      for j in range(_NCH):
        pltpu.sync_copy(vals_v.at[pl.ds(j * _CH, _CH)],
                        acc.at[idx_v.at[r * _NCH + j]], add=True)
    plsc.subcore_barrier()
    pltpu.sync_copy(acc.at[pl.ds(s * zpt, zpt)],
                    out_hbm.at[c, pl.ds(s * zpt, zpt)])

  return k(vals, idxs, zrows)


def _edge_tc(xs, ea8, wstack, extra8):
  be = 2048

  def body(xs_ref, ea_ref, w_ref, ex_ref, out_ref):
    xsb = xs_ref[...]
    acc = jnp.dot(xsb, w_ref[4], preferred_element_type=jnp.float32)
    for kk in range(4):
      zk = jnp.dot(xsb, w_ref[kk], preferred_element_type=jnp.float32)
      acc = acc + ea_ref[:, kk:kk + 1] * zk
    acc = acc + ex_ref[0:1, :]
    out_ref[...] = acc

  return pl.pallas_call(
      body,
      grid=(_EP // be,),
      in_specs=[
          pl.BlockSpec((be, _D), lambda i: (i, 0)),
          pl.BlockSpec((be, 8), lambda i: (i, 0)),
          pl.BlockSpec((5, _D, _D), lambda i: (0, 0, 0)),
          pl.BlockSpec((8, _D), lambda i: (0, 0)),
      ],
      out_specs=pl.BlockSpec((be, _D), lambda i: (i, 0)),
      out_shape=jax.ShapeDtypeStruct((_EP, _D), jnp.float32),
  )(xs, ea8, wstack, extra8)


def _node_tc(s3, rin16, x, root, bias8, mask8, first):
  bn = 2000

  def body(sr, rr16, xr, rootr, br, mr, outr, rout):
    s = sr[...]
    if first:
      cnt = s[:, 112:113]
      rin = 1.0 / jnp.maximum(cnt, 1.0)
    else:
      rin = rr16[:, 0:1]
    z = jnp.dot(xr[...], rootr[...], preferred_element_type=jnp.float32)
    h = jnp.maximum(s * rin + z + br[0:1, :], 0.0)
    outr[...] = h * mr[0:1, :]
    rout[...] = jnp.broadcast_to(rin, (bn, 16))

  return pl.pallas_call(
      body,
      grid=(_N // bn,),
      in_specs=[
          pl.BlockSpec((bn, _D), lambda i: (i, 0)),
          pl.BlockSpec((bn, 16), lambda i: (i, 0)),
          pl.BlockSpec((bn, _D), lambda i: (i, 0)),
          pl.BlockSpec((_D, _D), lambda i: (0, 0)),
          pl.BlockSpec((8, _D), lambda i: (0, 0)),
          pl.BlockSpec((8, _D), lambda i: (0, 0)),
      ],
      out_specs=[
          pl.BlockSpec((bn, _D), lambda i: (i, 0)),
          pl.BlockSpec((bn, 16), lambda i: (i, 0)),
      ],
      out_shape=[
          jax.ShapeDtypeStruct((_N, _D), jnp.float32),
          jax.ShapeDtypeStruct((_N, 16), jnp.float32),
      ],
  )(s3, rin16, x, root, bias8, mask8)


def _pool_fc_tc(h, batch, w1, b18, w2, b28):
  def body(b_ref, h_ref, w1_ref, b1_ref, w2_ref, b2_ref, out_ref, acc):
    acc[...] = jnp.zeros_like(acc)

    def step(i, c):
      g = b_ref[i]
      row = h_ref[pl.ds(i, 1), :]
      acc[pl.ds(g, 1), :] = jnp.maximum(acc[pl.ds(g, 1), :], row)
      return c

    lax.fori_loop(0, _N, step, 0)
    p = acc[0:_G, :]
    hh = jnp.maximum(
        jnp.dot(p, w1_ref[...], preferred_element_type=jnp.float32)
        + b1_ref[0:1, :], 0.0)
    out_ref[...] = jnp.dot(hh, w2_ref[...],
                           preferred_element_type=jnp.float32) + b2_ref[0:1, :]

  return pl.pallas_call(
      body,
      grid=(1,),
      in_specs=[
          pl.BlockSpec(memory_space=pltpu.SMEM),
          pl.BlockSpec((_N, _D), lambda i: (0, 0)),
          pl.BlockSpec((_D, 64), lambda i: (0, 0)),
          pl.BlockSpec((8, 64), lambda i: (0, 0)),
          pl.BlockSpec((64, 10), lambda i: (0, 0)),
          pl.BlockSpec((8, 10), lambda i: (0, 0)),
      ],
      out_specs=pl.BlockSpec((_G, 10), lambda i: (0, 0)),
      out_shape=jax.ShapeDtypeStruct((_G, 10), jnp.float32),
      scratch_shapes=[pltpu.VMEM((512, _D), jnp.float32)],
  )(batch, h, w1, b18, w2, b28)


def kernel(x, edge_attr, e1W, e1b, e2W, e2b, e3W, e3b, e4W, e4b,
           root1, bias1, root2, bias2, root3, bias3, root4, bias4,
           fc1W, fc1b, fc2W, fc2b, edge_index, batch):
  dims = [(128, 32), (32, 48), (48, 64), (64, 128)]
  eWs = [e1W, e2W, e3W, e4W]
  ebs = [e1b, e2b, e3b, e4b]
  roots = [root1, root2, root3, root4]
  biases = [bias1, bias2, bias3, bias4]

  src = edge_index[0]
  dst = edge_index[1]
  pad = _EP - _E
  src_p = jnp.concatenate([src, jnp.zeros((pad,), jnp.int32)])
  dst_p = jnp.concatenate([dst, jnp.full((pad,), _N, jnp.int32)])
  idx_src = src_p.reshape(_NW, _NCH, _CH)
  idx_h = jnp.stack([
      jnp.where(dst_p < _HROWS, dst_p, _HROWS),
      jnp.where(dst_p >= _HROWS, dst_p - _HROWS, _N - _HROWS),
  ]).reshape(2, 16, 2 * _NCH, _CH)
  ea8 = jnp.concatenate(
      [edge_attr, jnp.zeros((pad, 4), jnp.float32)], axis=0)
  ea8 = jnp.concatenate([ea8, jnp.zeros((_EP, 4), jnp.float32)], axis=1)
  zrows = jnp.zeros((_EPT, _D), jnp.float32)

  h = x
  rin16 = jnp.zeros((_N, 16), jnp.float32)
  for i, (ci, co) in enumerate(dims):
    wstack = jnp.zeros((5, _D, _D), jnp.float32).at[:, :ci, :co].set(
        jnp.concatenate(
            [eWs[i].reshape(4, ci, co), ebs[i].reshape(1, ci, co)], axis=0))
    rootp = jnp.zeros((_D, _D), jnp.float32).at[:ci, :co].set(roots[i])
    bias8 = jnp.broadcast_to(
        jnp.pad(biases[i], (0, _D - co)), (8, _D))
    mask8 = jnp.broadcast_to(
        (jnp.arange(_D) < co).astype(jnp.float32), (8, _D))
    extra8 = jnp.broadcast_to(
        ((jnp.arange(_D) >= 112) & (i == 0)).astype(jnp.float32), (8, _D))

    xs = _sc_gather(h, idx_src)
    msg = _edge_tc(xs, ea8, wstack, extra8)
    sh = _sc_scatter_add(msg, idx_h, zrows)
    s3 = jnp.concatenate([sh[0, :_HROWS], sh[1, :_HROWS]], axis=0)
    h, rin16 = _node_tc(s3, rin16, h, rootp, bias8, mask8, first=(i == 0))

  return _pool_fc_tc(h, batch, fc1W, jnp.broadcast_to(fc1b, (8, 64)),
                     fc2W, jnp.broadcast_to(fc2b, (8, 10)))
